# baseline (speedup 1.0000x reference)
import functools

import jax
import jax.numpy as jnp
from jax import lax
from jax.experimental import pallas as pl
from jax.experimental.pallas import tpu as pltpu
from jax.experimental.pallas import tpu_sc as plsc

CB = 8192
H = 256
TOK_BLK = 256
CHUNK = 1024
BIG = 2 ** 30


def _argmin_body(z_ref, embt_ref, idx_ref, sum_ref, e2_ref, dist_ref):
    i = pl.program_id(0)

    @pl.when(i == 0)
    def _():
        embt = embt_ref[...]
        e2_ref[...] = jnp.sum(embt * embt, axis=0, keepdims=True)

    zb = z_ref[...]
    a = (2.0 * zb).astype(jnp.bfloat16)
    z2 = jnp.sum(zb * zb, axis=1, keepdims=True)
    mins = []
    for c in range(0, CB, CHUNK):
        s2c = jnp.dot(a, embt_ref[:, c:c + CHUNK],
                      preferred_element_type=jnp.float32)
        dc = (z2 - s2c) + e2_ref[:, c:c + CHUNK]
        dist_ref[:, c:c + CHUNK] = dc
        mins.append(jnp.min(dc, axis=1, keepdims=True))
    sel_val = jnp.min(jnp.concatenate(mins, axis=1), axis=1)
    idxs = []
    for c in range(0, CB, CHUNK):
        dc = dist_ref[:, c:c + CHUNK]
        jc = c + lax.broadcasted_iota(jnp.int32, (TOK_BLK, CHUNK), 1)
        idxs.append(jnp.min(jnp.where(dc == sel_val[:, None], jc, BIG),
                            axis=1, keepdims=True))
    sel_idx = jnp.min(jnp.concatenate(idxs, axis=1), axis=1)
    idx_ref[0, 0, :] = sel_idx

    @pl.when(i == 0)
    def _():
        sum_ref[...] = jnp.zeros((1, 1), jnp.float32)

    sum_ref[...] += jnp.sum(sel_val).reshape(1, 1)


def _argmin_call(flat, embt, interpret=False):
    n_blk = flat.shape[0] // TOK_BLK
    return pl.pallas_call(
        _argmin_body,
        grid=(n_blk,),
        in_specs=[
            pl.BlockSpec((TOK_BLK, H), lambda i: (i, 0)),
            pl.BlockSpec((H, CB), lambda i: (0, 0)),
        ],
        out_specs=[
            pl.BlockSpec((1, 1, TOK_BLK), lambda i: (i, 0, 0)),
            pl.BlockSpec((1, 1), lambda i: (0, 0)),
        ],
        out_shape=[
            jax.ShapeDtypeStruct((n_blk, 1, TOK_BLK), jnp.int32),
            jax.ShapeDtypeStruct((1, 1), jnp.float32),
        ],
        scratch_shapes=[pltpu.VMEM((1, CB), jnp.float32),
                        pltpu.VMEM((TOK_BLK, CB), jnp.float32)],
        interpret=interpret,
    )(flat, embt)


def _make_sc_gather(B, D):
    info = plsc.get_sparse_core_info()
    NC, NS = info.num_cores, info.num_subcores
    NW = NC * NS
    b_per_w = B // NW
    CH = 128
    n_ch = b_per_w // CH
    mesh = plsc.VectorSubcoreMesh(core_axis_name="c", subcore_axis_name="s")

    @functools.partial(
        pl.kernel,
        mesh=mesh,
        out_type=jax.ShapeDtypeStruct((B, D), jnp.float32),
        scratch_types=[
            pltpu.VMEM((n_ch, CH), jnp.int32),
            pltpu.VMEM((CH, D), jnp.float32),
            pltpu.VMEM((CH, D), jnp.float32),
            pltpu.SemaphoreType.DMA,
            pltpu.SemaphoreType.DMA,
        ],
    )
    def gather(table_hbm, idx_hbm, out_hbm, idx_v, rows0, rows1, sem0, sem1):
        wid = lax.axis_index("s") * NC + lax.axis_index("c")
        base = wid * b_per_w
        pltpu.sync_copy(idx_hbm.at[wid], idx_v)
        bufs = (rows0, rows1)
        sems = (sem0, sem1)
        handles = [None, None]
        handles[0] = pltpu.async_copy(
            table_hbm.at[idx_v.at[0]], bufs[0], sems[0])
        for c in range(n_ch):
            if c + 1 < n_ch:
                handles[(c + 1) % 2] = pltpu.async_copy(
                    table_hbm.at[idx_v.at[c + 1]],
                    bufs[(c + 1) % 2], sems[(c + 1) % 2])
            handles[c % 2].wait()
            pltpu.sync_copy(bufs[c % 2],
                            out_hbm.at[pl.ds(base + c * CH, CH)])

    return gather, NW, n_ch, CH


NSPLIT = 2


def kernel(z, emb):
    b, t, h = z.shape
    n = b * t
    flat = z.reshape(n, h)
    embt = emb.T
    m = n // NSPLIT
    sc_gather, NW, n_ch, CH = _make_sc_gather(m, h)
    idx_parts, zq_parts, sums = [], [], []
    for s in range(NSPLIT):
        idx3, msum = _argmin_call(flat[s * m:(s + 1) * m], embt)
        idx_s = idx3.reshape(m)
        zq_parts.append(sc_gather(emb, idx_s.reshape(NW, n_ch, CH)))
        idx_parts.append(idx_s)
        sums.append(msum[0, 0])
    z_q = jnp.concatenate(zq_parts, axis=0)
    idx = jnp.concatenate(idx_parts, axis=0)
    loss = sum(sums) / (n * h)
    return (z_q.reshape(b, t, h), idx.reshape(b, t), loss, loss)


# speedup vs baseline: 1.0946x; 1.0946x over previous
import functools

import jax
import jax.numpy as jnp
from jax import lax
from jax.experimental import pallas as pl
from jax.experimental.pallas import tpu as pltpu
from jax.experimental.pallas import tpu_sc as plsc

CB = 8192
H = 256
TOK_BLK = 512
CHUNK = 1024
BIG = 2 ** 30


def _argmin_body(z_ref, embt_ref, idx_ref, sum_ref, e2_ref, dist_ref):
    i = pl.program_id(0)

    @pl.when(i == 0)
    def _():
        embt = embt_ref[...]
        e2_ref[...] = jnp.sum(embt * embt, axis=0, keepdims=True)

    zb = z_ref[...]
    a = (2.0 * zb).astype(jnp.bfloat16)
    z2 = jnp.sum(zb * zb, axis=1, keepdims=True)
    mins = []
    for c in range(0, CB, CHUNK):
        s2c = jnp.dot(a, embt_ref[:, c:c + CHUNK],
                      preferred_element_type=jnp.float32)
        dc = (z2 - s2c) + e2_ref[:, c:c + CHUNK]
        dist_ref[:, c:c + CHUNK] = dc
        mins.append(jnp.min(dc, axis=1, keepdims=True))
    sel_val = jnp.min(jnp.concatenate(mins, axis=1), axis=1)
    idxs = []
    for c in range(0, CB, CHUNK):
        dc = dist_ref[:, c:c + CHUNK]
        jc = c + lax.broadcasted_iota(jnp.int32, (TOK_BLK, CHUNK), 1)
        idxs.append(jnp.min(jnp.where(dc == sel_val[:, None], jc, BIG),
                            axis=1, keepdims=True))
    sel_idx = jnp.min(jnp.concatenate(idxs, axis=1), axis=1)
    idx_ref[0, 0, :] = sel_idx

    @pl.when(i == 0)
    def _():
        sum_ref[...] = jnp.zeros((1, 1), jnp.float32)

    sum_ref[...] += jnp.sum(sel_val).reshape(1, 1)


def _argmin_call(flat, embt, interpret=False):
    n_blk = flat.shape[0] // TOK_BLK
    return pl.pallas_call(
        _argmin_body,
        grid=(n_blk,),
        in_specs=[
            pl.BlockSpec((TOK_BLK, H), lambda i: (i, 0)),
            pl.BlockSpec((H, CB), lambda i: (0, 0)),
        ],
        out_specs=[
            pl.BlockSpec((1, 1, TOK_BLK), lambda i: (i, 0, 0)),
            pl.BlockSpec((1, 1), lambda i: (0, 0)),
        ],
        out_shape=[
            jax.ShapeDtypeStruct((n_blk, 1, TOK_BLK), jnp.int32),
            jax.ShapeDtypeStruct((1, 1), jnp.float32),
        ],
        scratch_shapes=[pltpu.VMEM((1, CB), jnp.float32),
                        pltpu.VMEM((TOK_BLK, CB), jnp.float32)],
        interpret=interpret,
    )(flat, embt)


def _make_sc_gather(B, D):
    info = plsc.get_sparse_core_info()
    NC, NS = info.num_cores, info.num_subcores
    NW = NC * NS
    b_per_w = B // NW
    CH = 128
    n_ch = b_per_w // CH
    mesh = plsc.VectorSubcoreMesh(core_axis_name="c", subcore_axis_name="s")

    @functools.partial(
        pl.kernel,
        mesh=mesh,
        out_type=jax.ShapeDtypeStruct((B, D), jnp.float32),
        scratch_types=[
            pltpu.VMEM((n_ch, CH), jnp.int32),
            pltpu.VMEM((CH, D), jnp.float32),
            pltpu.VMEM((CH, D), jnp.float32),
            pltpu.SemaphoreType.DMA,
            pltpu.SemaphoreType.DMA,
        ],
    )
    def gather(table_hbm, idx_hbm, out_hbm, idx_v, rows0, rows1, sem0, sem1):
        wid = lax.axis_index("s") * NC + lax.axis_index("c")
        base = wid * b_per_w
        pltpu.sync_copy(idx_hbm.at[wid], idx_v)
        bufs = (rows0, rows1)
        sems = (sem0, sem1)
        handles = [None, None]
        handles[0] = pltpu.async_copy(
            table_hbm.at[idx_v.at[0]], bufs[0], sems[0])
        for c in range(n_ch):
            if c + 1 < n_ch:
                handles[(c + 1) % 2] = pltpu.async_copy(
                    table_hbm.at[idx_v.at[c + 1]],
                    bufs[(c + 1) % 2], sems[(c + 1) % 2])
            handles[c % 2].wait()
            pltpu.sync_copy(bufs[c % 2],
                            out_hbm.at[pl.ds(base + c * CH, CH)])

    return gather, NW, n_ch, CH


def kernel(z, emb):
    b, t, h = z.shape
    n = b * t
    flat = z.reshape(n, h)
    embt = emb.T
    idx3, msum = _argmin_call(flat, embt)
    idx = idx3.reshape(n)
    sc_gather, NW, n_ch, CH = _make_sc_gather(n, h)
    z_q = sc_gather(emb, idx.reshape(NW, n_ch, CH))
    loss = msum[0, 0] / (n * h)
    return (z_q.reshape(b, t, h), idx.reshape(b, t), loss, loss)


# speedup vs baseline: 1.1156x; 1.0191x over previous
import functools

import jax
import jax.numpy as jnp
from jax import lax
from jax.experimental import pallas as pl
from jax.experimental.pallas import tpu as pltpu
from jax.experimental.pallas import tpu_sc as plsc

CB = 8192
H = 256
TOK_BLK = 1024
CHUNK = 1024
BIG = 2 ** 30


def _argmin_body(z_ref, embt_ref, idx_ref, sum_ref, e2_ref, dist_ref):
    i = pl.program_id(0)

    @pl.when(i == 0)
    def _():
        embt = embt_ref[...]
        e2_ref[...] = jnp.sum(embt * embt, axis=0, keepdims=True)

    zb = z_ref[...]
    a = (2.0 * zb).astype(jnp.bfloat16)
    z2 = jnp.sum(zb * zb, axis=1, keepdims=True)
    mins = []
    for c in range(0, CB, CHUNK):
        s2c = jnp.dot(a, embt_ref[:, c:c + CHUNK],
                      preferred_element_type=jnp.float32)
        dc = (z2 - s2c) + e2_ref[:, c:c + CHUNK]
        dist_ref[:, c:c + CHUNK] = dc
        mins.append(jnp.min(dc, axis=1, keepdims=True))
    sel_val = jnp.min(jnp.concatenate(mins, axis=1), axis=1)
    idxs = []
    for c in range(0, CB, CHUNK):
        dc = dist_ref[:, c:c + CHUNK]
        jc = c + lax.broadcasted_iota(jnp.int32, (TOK_BLK, CHUNK), 1)
        idxs.append(jnp.min(jnp.where(dc == sel_val[:, None], jc, BIG),
                            axis=1, keepdims=True))
    sel_idx = jnp.min(jnp.concatenate(idxs, axis=1), axis=1)
    idx_ref[0, 0, :] = sel_idx

    @pl.when(i == 0)
    def _():
        sum_ref[...] = jnp.zeros((1, 1), jnp.float32)

    sum_ref[...] += jnp.sum(sel_val).reshape(1, 1)


def _argmin_call(flat, embt, interpret=False):
    n_blk = flat.shape[0] // TOK_BLK
    return pl.pallas_call(
        _argmin_body,
        grid=(n_blk,),
        in_specs=[
            pl.BlockSpec((TOK_BLK, H), lambda i: (i, 0)),
            pl.BlockSpec((H, CB), lambda i: (0, 0)),
        ],
        out_specs=[
            pl.BlockSpec((1, 1, TOK_BLK), lambda i: (i, 0, 0)),
            pl.BlockSpec((1, 1), lambda i: (0, 0)),
        ],
        out_shape=[
            jax.ShapeDtypeStruct((n_blk, 1, TOK_BLK), jnp.int32),
            jax.ShapeDtypeStruct((1, 1), jnp.float32),
        ],
        scratch_shapes=[pltpu.VMEM((1, CB), jnp.float32),
                        pltpu.VMEM((TOK_BLK, CB), jnp.float32)],
        interpret=interpret,
    )(flat, embt)


def _make_sc_gather(B, D):
    info = plsc.get_sparse_core_info()
    NC, NS = info.num_cores, info.num_subcores
    NW = NC * NS
    b_per_w = B // NW
    CH = 128
    n_ch = b_per_w // CH
    mesh = plsc.VectorSubcoreMesh(core_axis_name="c", subcore_axis_name="s")

    @functools.partial(
        pl.kernel,
        mesh=mesh,
        out_type=jax.ShapeDtypeStruct((B, D), jnp.float32),
        scratch_types=[
            pltpu.VMEM((n_ch, CH), jnp.int32),
            pltpu.VMEM((CH, D), jnp.float32),
            pltpu.VMEM((CH, D), jnp.float32),
            pltpu.SemaphoreType.DMA,
            pltpu.SemaphoreType.DMA,
        ],
    )
    def gather(table_hbm, idx_hbm, out_hbm, idx_v, rows0, rows1, sem0, sem1):
        wid = lax.axis_index("s") * NC + lax.axis_index("c")
        base = wid * b_per_w
        pltpu.sync_copy(idx_hbm.at[wid], idx_v)
        bufs = (rows0, rows1)
        sems = (sem0, sem1)
        handles = [None, None]
        handles[0] = pltpu.async_copy(
            table_hbm.at[idx_v.at[0]], bufs[0], sems[0])
        for c in range(n_ch):
            if c + 1 < n_ch:
                handles[(c + 1) % 2] = pltpu.async_copy(
                    table_hbm.at[idx_v.at[c + 1]],
                    bufs[(c + 1) % 2], sems[(c + 1) % 2])
            handles[c % 2].wait()
            pltpu.sync_copy(bufs[c % 2],
                            out_hbm.at[pl.ds(base + c * CH, CH)])

    return gather, NW, n_ch, CH


def kernel(z, emb):
    b, t, h = z.shape
    n = b * t
    flat = z.reshape(n, h)
    embt = emb.T
    idx3, msum = _argmin_call(flat, embt)
    idx = idx3.reshape(n)
    sc_gather, NW, n_ch, CH = _make_sc_gather(n, h)
    z_q = sc_gather(emb, idx.reshape(NW, n_ch, CH))
    loss = msum[0, 0] / (n * h)
    return (z_q.reshape(b, t, h), idx.reshape(b, t), loss, loss)


# speedup vs baseline: 1.1296x; 1.0126x over previous
import functools

import jax
import jax.numpy as jnp
from jax import lax
from jax.experimental import pallas as pl
from jax.experimental.pallas import tpu as pltpu
from jax.experimental.pallas import tpu_sc as plsc

CB = 8192
H = 256
TOK_BLK = 1024
CHUNK = 2048
BIG = 2 ** 30


def _argmin_body(z_ref, embt_ref, idx_ref, sum_ref, e2_ref, dist_ref):
    i = pl.program_id(0)

    @pl.when(i == 0)
    def _():
        embt = embt_ref[...]
        e2_ref[...] = jnp.sum(embt * embt, axis=0, keepdims=True)

    zb = z_ref[...]
    a = (2.0 * zb).astype(jnp.bfloat16)
    z2 = jnp.sum(zb * zb, axis=1, keepdims=True)
    mins = []
    for c in range(0, CB, CHUNK):
        s2c = jnp.dot(a, embt_ref[:, c:c + CHUNK],
                      preferred_element_type=jnp.float32)
        dc = (z2 - s2c) + e2_ref[:, c:c + CHUNK]
        dist_ref[:, c:c + CHUNK] = dc
        mins.append(jnp.min(dc, axis=1, keepdims=True))
    sel_val = jnp.min(jnp.concatenate(mins, axis=1), axis=1)
    idxs = []
    for c in range(0, CB, CHUNK):
        dc = dist_ref[:, c:c + CHUNK]
        jc = c + lax.broadcasted_iota(jnp.int32, (TOK_BLK, CHUNK), 1)
        idxs.append(jnp.min(jnp.where(dc == sel_val[:, None], jc, BIG),
                            axis=1, keepdims=True))
    sel_idx = jnp.min(jnp.concatenate(idxs, axis=1), axis=1)
    idx_ref[0, 0, :] = sel_idx

    @pl.when(i == 0)
    def _():
        sum_ref[...] = jnp.zeros((1, 1), jnp.float32)

    sum_ref[...] += jnp.sum(sel_val).reshape(1, 1)


def _argmin_call(flat, embt, interpret=False):
    n_blk = flat.shape[0] // TOK_BLK
    return pl.pallas_call(
        _argmin_body,
        grid=(n_blk,),
        in_specs=[
            pl.BlockSpec((TOK_BLK, H), lambda i: (i, 0)),
            pl.BlockSpec((H, CB), lambda i: (0, 0)),
        ],
        out_specs=[
            pl.BlockSpec((1, 1, TOK_BLK), lambda i: (i, 0, 0)),
            pl.BlockSpec((1, 1), lambda i: (0, 0)),
        ],
        out_shape=[
            jax.ShapeDtypeStruct((n_blk, 1, TOK_BLK), jnp.int32),
            jax.ShapeDtypeStruct((1, 1), jnp.float32),
        ],
        scratch_shapes=[pltpu.VMEM((1, CB), jnp.float32),
                        pltpu.VMEM((TOK_BLK, CB), jnp.float32)],
        interpret=interpret,
    )(flat, embt)


def _make_sc_gather(B, D):
    info = plsc.get_sparse_core_info()
    NC, NS = info.num_cores, info.num_subcores
    NW = NC * NS
    b_per_w = B // NW
    CH = 128
    n_ch = b_per_w // CH
    mesh = plsc.VectorSubcoreMesh(core_axis_name="c", subcore_axis_name="s")

    @functools.partial(
        pl.kernel,
        mesh=mesh,
        out_type=jax.ShapeDtypeStruct((B, D), jnp.float32),
        scratch_types=[
            pltpu.VMEM((n_ch, CH), jnp.int32),
            pltpu.VMEM((CH, D), jnp.float32),
            pltpu.VMEM((CH, D), jnp.float32),
            pltpu.SemaphoreType.DMA,
            pltpu.SemaphoreType.DMA,
        ],
    )
    def gather(table_hbm, idx_hbm, out_hbm, idx_v, rows0, rows1, sem0, sem1):
        wid = lax.axis_index("s") * NC + lax.axis_index("c")
        base = wid * b_per_w
        pltpu.sync_copy(idx_hbm.at[wid], idx_v)
        bufs = (rows0, rows1)
        sems = (sem0, sem1)
        handles = [None, None]
        handles[0] = pltpu.async_copy(
            table_hbm.at[idx_v.at[0]], bufs[0], sems[0])
        for c in range(n_ch):
            if c + 1 < n_ch:
                handles[(c + 1) % 2] = pltpu.async_copy(
                    table_hbm.at[idx_v.at[c + 1]],
                    bufs[(c + 1) % 2], sems[(c + 1) % 2])
            handles[c % 2].wait()
            pltpu.sync_copy(bufs[c % 2],
                            out_hbm.at[pl.ds(base + c * CH, CH)])

    return gather, NW, n_ch, CH


def kernel(z, emb):
    b, t, h = z.shape
    n = b * t
    flat = z.reshape(n, h)
    embt = emb.T
    idx3, msum = _argmin_call(flat, embt)
    idx = idx3.reshape(n)
    sc_gather, NW, n_ch, CH = _make_sc_gather(n, h)
    z_q = sc_gather(emb, idx.reshape(NW, n_ch, CH))
    loss = msum[0, 0] / (n * h)
    return (z_q.reshape(b, t, h), idx.reshape(b, t), loss, loss)
